# baseline (speedup 1.0000x reference)
import jax
import jax.numpy as jnp
from jax.experimental import pallas as pl
from jax.experimental.pallas import tpu as pltpu

B, S, H = 16, 4096, 1024
D = 64
W = 2 * D
WP = W + 16
f32 = jnp.float32
bf16 = jnp.bfloat16


def _pt_kernel(yt_ref, len_ref, wtan_ref, wpt_ref,
               start_ref, left_ref, right_ref, pt_ref):
    yt16 = yt_ref[:].astype(bf16)
    z1 = jnp.dot(yt16, wtan_ref[:].astype(bf16), preferred_element_type=f32)
    t16 = jnp.tanh(z1).astype(bf16)
    logit = jnp.dot(t16, wpt_ref[:].astype(bf16), preferred_element_type=f32)
    lens_i = len_ref[:]
    pt = jax.nn.sigmoid(logit) * lens_i.astype(f32)
    pti = jnp.floor(pt).astype(jnp.int32)
    left = jnp.maximum(0, pti - D)
    right = jnp.minimum(lens_i, pti + D)
    start = jnp.minimum((left // 16) * 16, S - WP)
    start_ref[:] = start
    left_ref[:] = left
    right_ref[:] = right
    pt_ref[:] = pt


def _attn_kernel(start_smem, hbm_ref, yt_ref, pt_ref, start_ref, left_ref,
                 right_ref, wct_ref, out_ref, gath, sems):
    copies = []
    for b in range(B):
        s = pl.multiple_of(start_smem[0, b], 16)
        cp = pltpu.make_async_copy(
            hbm_ref.at[b, pl.ds(s, WP), :], gath.at[b], sems.at[b])
        cp.start()
        copies.append(cp)
    for cp in copies:
        cp.wait()

    g = gath[:].astype(f32)
    yt16 = yt_ref[:].astype(bf16).astype(f32)
    scores = jnp.sum(g * yt16[:, None, :], axis=2)

    jpos = start_ref[:] + jax.lax.broadcasted_iota(jnp.int32, (B, WP), 1)
    valid = (jpos >= left_ref[:]) & (jpos < right_ref[:])
    scores = jnp.where(valid, scores, -1e30)
    m = jnp.max(scores, axis=1, keepdims=True)
    e = jnp.exp(scores - m)
    align = e / jnp.sum(e, axis=1, keepdims=True)
    pt = pt_ref[:]
    d = jpos.astype(f32) - pt
    ex_p = jnp.exp(-(d * d) / (D * D / 2.0))
    at = (align * ex_p * valid.astype(f32)).astype(bf16).astype(f32)

    ct = jnp.sum(at[:, :, None] * g, axis=1).astype(bf16)
    out_ref[:] = jnp.dot(ct, wct_ref[:].astype(bf16), preferred_element_type=f32)


def kernel(encode_h, yt, encode_len, W_tan, w_pt, W_ct2ht):
    len2d = encode_len.reshape(B, 1)
    start, left, right, pt = pl.pallas_call(
        _pt_kernel,
        out_shape=(
            jax.ShapeDtypeStruct((B, 1), jnp.int32),
            jax.ShapeDtypeStruct((B, 1), jnp.int32),
            jax.ShapeDtypeStruct((B, 1), jnp.int32),
            jax.ShapeDtypeStruct((B, 1), f32),
        ),
    )(yt, len2d, W_tan, w_pt)

    ht = pl.pallas_call(
        _attn_kernel,
        in_specs=[
            pl.BlockSpec(memory_space=pltpu.MemorySpace.SMEM),
            pl.BlockSpec(memory_space=pl.ANY),
            pl.BlockSpec(memory_space=pltpu.MemorySpace.VMEM),
            pl.BlockSpec(memory_space=pltpu.MemorySpace.VMEM),
            pl.BlockSpec(memory_space=pltpu.MemorySpace.VMEM),
            pl.BlockSpec(memory_space=pltpu.MemorySpace.VMEM),
            pl.BlockSpec(memory_space=pltpu.MemorySpace.VMEM),
            pl.BlockSpec(memory_space=pltpu.MemorySpace.VMEM),
        ],
        out_shape=jax.ShapeDtypeStruct((B, H), f32),
        scratch_shapes=[
            pltpu.VMEM((B, WP, H), bf16),
            pltpu.SemaphoreType.DMA((B,)),
        ],
    )(start.reshape(1, B), encode_h.astype(bf16), yt, pt, start, left, right,
      W_ct2ht)
    return ht


# speedup vs baseline: 7.9194x; 7.9194x over previous
import jax
import jax.numpy as jnp
from jax.experimental import pallas as pl
from jax.experimental.pallas import tpu as pltpu

B, S, H = 16, 4096, 1024
D = 64
W = 2 * D
WP = W + 8
f32 = jnp.float32
bf16 = jnp.bfloat16


def _pt_kernel(yt_ref, len_ref, wtan_ref, wpt_ref,
               start_ref, left_ref, right_ref, pt_ref):
    yt16 = yt_ref[:].astype(bf16)
    z1 = jnp.dot(yt16, wtan_ref[:].astype(bf16), preferred_element_type=f32)
    t16 = jnp.tanh(z1).astype(bf16)
    logit = jnp.dot(t16, wpt_ref[:].astype(bf16), preferred_element_type=f32)
    lens_i = len_ref[:]
    pt = jax.nn.sigmoid(logit) * lens_i.astype(f32)
    pti = jnp.floor(pt).astype(jnp.int32)
    left = jnp.maximum(0, pti - D)
    right = jnp.minimum(lens_i, pti + D)
    start = jnp.minimum((left // 8) * 8, S - WP)
    start_ref[:] = start
    left_ref[:] = left
    right_ref[:] = right
    pt_ref[:] = pt


def _attn_kernel(start_smem, hbm_ref, yt_ref, pt_ref, start_ref, left_ref,
                 right_ref, wct_ref, out_ref, gath, sems):
    copies = []
    for b in range(B):
        s = pl.multiple_of(start_smem[0, b], 8)
        cp = pltpu.make_async_copy(
            hbm_ref.at[b, pl.ds(s, WP), :], gath.at[b], sems.at[b])
        cp.start()
        copies.append(cp)
    for cp in copies:
        cp.wait()

    g = gath[:].astype(bf16).astype(f32)
    yt16 = yt_ref[:].astype(bf16).astype(f32)
    scores = jnp.sum(g * yt16[:, None, :], axis=2)

    jpos = start_ref[:] + jax.lax.broadcasted_iota(jnp.int32, (B, WP), 1)
    valid = (jpos >= left_ref[:]) & (jpos < right_ref[:])
    scores = jnp.where(valid, scores, -1e30)
    m = jnp.max(scores, axis=1, keepdims=True)
    e = jnp.exp(scores - m)
    align = e / jnp.sum(e, axis=1, keepdims=True)
    pt = pt_ref[:]
    d = jpos.astype(f32) - pt
    ex_p = jnp.exp(-(d * d) / (D * D / 2.0))
    at = (align * ex_p * valid.astype(f32)).astype(bf16).astype(f32)

    ct = jnp.sum(at[:, :, None] * g, axis=1).astype(bf16)
    out_ref[:] = jnp.dot(ct, wct_ref[:].astype(bf16), preferred_element_type=f32)


def kernel(encode_h, yt, encode_len, W_tan, w_pt, W_ct2ht):
    len2d = encode_len.reshape(B, 1)
    start, left, right, pt = pl.pallas_call(
        _pt_kernel,
        out_shape=(
            jax.ShapeDtypeStruct((B, 1), jnp.int32),
            jax.ShapeDtypeStruct((B, 1), jnp.int32),
            jax.ShapeDtypeStruct((B, 1), jnp.int32),
            jax.ShapeDtypeStruct((B, 1), f32),
        ),
    )(yt, len2d, W_tan, w_pt)

    ht = pl.pallas_call(
        _attn_kernel,
        in_specs=[
            pl.BlockSpec(memory_space=pltpu.MemorySpace.SMEM),
            pl.BlockSpec(memory_space=pl.ANY),
            pl.BlockSpec(memory_space=pltpu.MemorySpace.VMEM),
            pl.BlockSpec(memory_space=pltpu.MemorySpace.VMEM),
            pl.BlockSpec(memory_space=pltpu.MemorySpace.VMEM),
            pl.BlockSpec(memory_space=pltpu.MemorySpace.VMEM),
            pl.BlockSpec(memory_space=pltpu.MemorySpace.VMEM),
            pl.BlockSpec(memory_space=pltpu.MemorySpace.VMEM),
        ],
        out_shape=jax.ShapeDtypeStruct((B, H), f32),
        scratch_shapes=[
            pltpu.VMEM((B, WP, H), f32),
            pltpu.SemaphoreType.DMA((B,)),
        ],
    )(start.reshape(1, B), encode_h, yt, pt, start, left, right, W_ct2ht)
    return ht


# speedup vs baseline: 9.2182x; 1.1640x over previous
import jax
import jax.numpy as jnp
from jax.experimental import pallas as pl
from jax.experimental.pallas import tpu as pltpu

B, S, H = 16, 4096, 1024
D = 64
W = 2 * D
WP = W + 8
f32 = jnp.float32
bf16 = jnp.bfloat16


def _nmt_kernel(hbm_ref, yt_ref, len_ref, wtan_ref, wpt_ref, wct_ref,
                out_ref, gath, startv, starts, sem, gsems):
    yt16 = yt_ref[:].astype(bf16)
    z1 = jnp.dot(yt16, wtan_ref[:].astype(bf16), preferred_element_type=f32)
    t16 = jnp.tanh(z1).astype(bf16)
    logit = jnp.dot(t16, wpt_ref[:].astype(bf16), preferred_element_type=f32)
    lens_i = len_ref[:]
    pt = jax.nn.sigmoid(logit) * lens_i.astype(f32)
    pti = jnp.floor(pt).astype(jnp.int32)
    left = jnp.maximum(0, pti - D)
    right = jnp.minimum(lens_i, pti + D)
    start = jnp.minimum((left // 8) * 8, S - WP)

    startv[:] = start
    cp = pltpu.make_async_copy(startv, starts, sem)
    cp.start()
    cp.wait()

    copies = []
    for b in range(B):
        s = pl.multiple_of(starts[b, 0], 8)
        c = pltpu.make_async_copy(
            hbm_ref.at[b, pl.ds(s, WP), :], gath.at[b], gsems.at[b])
        c.start()
        copies.append(c)
    for c in copies:
        c.wait()

    g = gath[:].astype(bf16).astype(f32)
    ytf = yt16.astype(f32)
    scores = jnp.sum(g * ytf[:, None, :], axis=2)

    jpos = start + jax.lax.broadcasted_iota(jnp.int32, (B, WP), 1)
    valid = (jpos >= left) & (jpos < right)
    scores = jnp.where(valid, scores, -1e30)
    m = jnp.max(scores, axis=1, keepdims=True)
    e = jnp.exp(scores - m)
    align = e / jnp.sum(e, axis=1, keepdims=True)
    d = jpos.astype(f32) - pt
    ex_p = jnp.exp(-(d * d) / (D * D / 2.0))
    at = (align * ex_p * valid.astype(f32)).astype(bf16).astype(f32)

    ct = jnp.sum(at[:, :, None] * g, axis=1).astype(bf16)
    out_ref[:] = jnp.dot(ct, wct_ref[:].astype(bf16), preferred_element_type=f32)


def kernel(encode_h, yt, encode_len, W_tan, w_pt, W_ct2ht):
    return pl.pallas_call(
        _nmt_kernel,
        in_specs=[
            pl.BlockSpec(memory_space=pl.ANY),
            pl.BlockSpec(memory_space=pltpu.MemorySpace.VMEM),
            pl.BlockSpec(memory_space=pltpu.MemorySpace.VMEM),
            pl.BlockSpec(memory_space=pltpu.MemorySpace.VMEM),
            pl.BlockSpec(memory_space=pltpu.MemorySpace.VMEM),
            pl.BlockSpec(memory_space=pltpu.MemorySpace.VMEM),
        ],
        out_shape=jax.ShapeDtypeStruct((B, H), f32),
        scratch_shapes=[
            pltpu.VMEM((B, WP, H), f32),
            pltpu.VMEM((B, 1), jnp.int32),
            pltpu.SMEM((B, 1), jnp.int32),
            pltpu.SemaphoreType.DMA,
            pltpu.SemaphoreType.DMA((B,)),
        ],
    )(encode_h, yt, encode_len.reshape(B, 1), W_tan, w_pt, W_ct2ht)


# speedup vs baseline: 10.0088x; 1.0858x over previous
import jax
import jax.numpy as jnp
from jax.experimental import pallas as pl
from jax.experimental.pallas import tpu as pltpu

B, S, H = 16, 4096, 1024
D = 64
W = 2 * D
WP = W + 8
f32 = jnp.float32
bf16 = jnp.bfloat16


def _nmt_kernel(hbm_ref, yt_ref, len_ref, wtan_ref, wpt_ref, wct_hbm,
                out_ref, gath, startv, starts, wct_vmem, sem, wsem, gsems):
    wcp = pltpu.make_async_copy(wct_hbm, wct_vmem, wsem)
    wcp.start()

    yt16 = yt_ref[:].astype(bf16)
    z1 = jnp.dot(yt16, wtan_ref[:].astype(bf16), preferred_element_type=f32)
    t16 = jnp.tanh(z1).astype(bf16)
    logit = jnp.dot(t16, wpt_ref[:].astype(bf16), preferred_element_type=f32)
    lens_i = len_ref[:]
    pt = jax.nn.sigmoid(logit) * lens_i.astype(f32)
    pti = jnp.floor(pt).astype(jnp.int32)
    left = jnp.maximum(0, pti - D)
    right = jnp.minimum(lens_i, pti + D)
    start = jnp.minimum((left // 8) * 8, S - WP)

    startv[:] = start
    cp = pltpu.make_async_copy(startv, starts, sem)
    cp.start()
    cp.wait()

    copies = []
    for b in range(B):
        s = pl.multiple_of(starts[b, 0], 8)
        c = pltpu.make_async_copy(
            hbm_ref.at[b, pl.ds(s, WP), :], gath.at[b], gsems.at[b])
        c.start()
        copies.append(c)
    for c in copies:
        c.wait()

    g16 = gath[:].astype(bf16)
    scores = jax.lax.dot_general(
        g16, yt16, (((2,), (1,)), ((0,), (0,))),
        preferred_element_type=f32)

    jpos = start + jax.lax.broadcasted_iota(jnp.int32, (B, WP), 1)
    valid = (jpos >= left) & (jpos < right)
    scores = jnp.where(valid, scores, -1e30)
    m = jnp.max(scores, axis=1, keepdims=True)
    e = jnp.exp(scores - m)
    align = e / jnp.sum(e, axis=1, keepdims=True)
    d = jpos.astype(f32) - pt
    ex_p = jnp.exp(-(d * d) / (D * D / 2.0))
    at16 = (align * ex_p * valid.astype(f32)).astype(bf16)

    ct = jax.lax.dot_general(
        at16, g16, (((1,), (1,)), ((0,), (0,))),
        preferred_element_type=f32).astype(bf16)
    wcp.wait()
    out_ref[:] = jnp.dot(ct, wct_vmem[:].astype(bf16),
                         preferred_element_type=f32)


def kernel(encode_h, yt, encode_len, W_tan, w_pt, W_ct2ht):
    return pl.pallas_call(
        _nmt_kernel,
        in_specs=[
            pl.BlockSpec(memory_space=pl.ANY),
            pl.BlockSpec(memory_space=pltpu.MemorySpace.VMEM),
            pl.BlockSpec(memory_space=pltpu.MemorySpace.VMEM),
            pl.BlockSpec(memory_space=pltpu.MemorySpace.VMEM),
            pl.BlockSpec(memory_space=pltpu.MemorySpace.VMEM),
            pl.BlockSpec(memory_space=pl.ANY),
        ],
        out_shape=jax.ShapeDtypeStruct((B, H), f32),
        scratch_shapes=[
            pltpu.VMEM((B, WP, H), f32),
            pltpu.VMEM((B, 1), jnp.int32),
            pltpu.SMEM((B, 1), jnp.int32),
            pltpu.VMEM((H, H), f32),
            pltpu.SemaphoreType.DMA,
            pltpu.SemaphoreType.DMA,
            pltpu.SemaphoreType.DMA((B,)),
        ],
    )(encode_h, yt, encode_len.reshape(B, 1), W_tan, w_pt, W_ct2ht)


# speedup vs baseline: 10.6773x; 1.0668x over previous
import jax
import jax.numpy as jnp
from jax.experimental import pallas as pl
from jax.experimental.pallas import tpu as pltpu

B, S, H = 16, 4096, 1024
D = 64
W = 2 * D
WP = W + 8
f32 = jnp.float32
bf16 = jnp.bfloat16


def _nmt_kernel(hbm_ref, yt_ref, len_ref, wtan_ref, wpt_ref, wct_hbm,
                out_ref, gath, startv, starts, wct_vmem, sem, wsem, gsems):
    wcp = pltpu.make_async_copy(wct_hbm, wct_vmem, wsem)
    wcp.start()

    yt16 = yt_ref[:].astype(bf16)
    z1 = jnp.dot(yt16, wtan_ref[:].astype(bf16), preferred_element_type=f32)
    t16 = jnp.tanh(z1).astype(bf16)
    logit = jnp.dot(t16, wpt_ref[:].astype(bf16), preferred_element_type=f32)
    lens_i = len_ref[:]
    pt = jax.nn.sigmoid(logit) * lens_i.astype(f32)
    pti = jnp.floor(pt).astype(jnp.int32)
    left = jnp.maximum(0, pti - D)
    right = jnp.minimum(lens_i, pti + D)
    start = jnp.minimum((left // 8) * 8, S - WP)

    startv[:] = start
    cp = pltpu.make_async_copy(startv, starts, sem)
    cp.start()
    cp.wait()

    copies = []
    for b in range(B):
        s = pl.multiple_of(starts[b, 0], 8)
        c = pltpu.make_async_copy(
            hbm_ref.at[b, pl.ds(s, WP), :], gath.at[b], gsems.at[b])
        c.start()
        copies.append(c)

    G = 4
    score_parts, g16_parts = [], []
    for gi in range(0, B, G):
        for b in range(gi, gi + G):
            copies[b].wait()
        gp16 = gath[gi:gi + G].astype(bf16)
        g16_parts.append(gp16)
        score_parts.append(jax.lax.dot_general(
            gp16, yt16[gi:gi + G], (((2,), (1,)), ((0,), (0,))),
            preferred_element_type=f32))
    scores = jnp.concatenate(score_parts, axis=0)
    g16 = jnp.concatenate(g16_parts, axis=0)

    jpos = start + jax.lax.broadcasted_iota(jnp.int32, (B, WP), 1)
    valid = (jpos >= left) & (jpos < right)
    scores = jnp.where(valid, scores, -1e30)
    m = jnp.max(scores, axis=1, keepdims=True)
    e = jnp.exp(scores - m)
    align = e / jnp.sum(e, axis=1, keepdims=True)
    d = jpos.astype(f32) - pt
    ex_p = jnp.exp(-(d * d) / (D * D / 2.0))
    at16 = (align * ex_p * valid.astype(f32)).astype(bf16)

    ct = jax.lax.dot_general(
        at16, g16, (((1,), (1,)), ((0,), (0,))),
        preferred_element_type=f32).astype(bf16)
    wcp.wait()
    out_ref[:] = jnp.dot(ct, wct_vmem[:].astype(bf16),
                         preferred_element_type=f32)


def kernel(encode_h, yt, encode_len, W_tan, w_pt, W_ct2ht):
    return pl.pallas_call(
        _nmt_kernel,
        in_specs=[
            pl.BlockSpec(memory_space=pl.ANY),
            pl.BlockSpec(memory_space=pltpu.MemorySpace.VMEM),
            pl.BlockSpec(memory_space=pltpu.MemorySpace.VMEM),
            pl.BlockSpec(memory_space=pltpu.MemorySpace.VMEM),
            pl.BlockSpec(memory_space=pltpu.MemorySpace.VMEM),
            pl.BlockSpec(memory_space=pl.ANY),
        ],
        out_shape=jax.ShapeDtypeStruct((B, H), f32),
        scratch_shapes=[
            pltpu.VMEM((B, WP, H), f32),
            pltpu.VMEM((B, 1), jnp.int32),
            pltpu.SMEM((B, 1), jnp.int32),
            pltpu.VMEM((H, H), f32),
            pltpu.SemaphoreType.DMA,
            pltpu.SemaphoreType.DMA,
            pltpu.SemaphoreType.DMA((B,)),
        ],
    )(encode_h, yt, encode_len.reshape(B, 1), W_tan, w_pt, W_ct2ht)


# speedup vs baseline: 10.6814x; 1.0004x over previous
import jax
import jax.numpy as jnp
from jax.experimental import pallas as pl
from jax.experimental.pallas import tpu as pltpu

B, S, H = 16, 4096, 1024
D = 64
W = 2 * D
WP = W + 8
f32 = jnp.float32
bf16 = jnp.bfloat16


def _nmt_kernel(hbm_ref, yt_ref, len_ref, wtan_ref, wpt_ref, wct_hbm,
                out_ref, gath, startv, starts, wct_vmem, sem, wsem, gsems):
    wcp = pltpu.make_async_copy(wct_hbm, wct_vmem, wsem)
    wcp.start()

    yt16 = yt_ref[:].astype(bf16)
    z1 = jnp.dot(yt16, wtan_ref[:].astype(bf16), preferred_element_type=f32)
    t16 = jnp.tanh(z1).astype(bf16)
    logit = jnp.dot(t16, wpt_ref[:].astype(bf16), preferred_element_type=f32)
    lens_i = len_ref[:]
    pt = jax.nn.sigmoid(logit) * lens_i.astype(f32)
    pti = jnp.floor(pt).astype(jnp.int32)
    left = jnp.maximum(0, pti - D)
    right = jnp.minimum(lens_i, pti + D)
    start = jnp.minimum((left // 8) * 8, S - WP)

    startv[:] = start

    copies = []
    for b in range(B):
        s = pl.multiple_of(startv[b, 0], 8)
        c = pltpu.make_async_copy(
            hbm_ref.at[b, pl.ds(s, WP), :], gath.at[b], gsems.at[b])
        c.start()
        copies.append(c)

    G = 4
    score_parts, g16_parts = [], []
    for gi in range(0, B, G):
        for b in range(gi, gi + G):
            copies[b].wait()
        gp16 = gath[gi:gi + G].astype(bf16)
        g16_parts.append(gp16)
        score_parts.append(jax.lax.dot_general(
            gp16, yt16[gi:gi + G], (((2,), (1,)), ((0,), (0,))),
            preferred_element_type=f32))
    scores = jnp.concatenate(score_parts, axis=0)
    g16 = jnp.concatenate(g16_parts, axis=0)

    jpos = start + jax.lax.broadcasted_iota(jnp.int32, (B, WP), 1)
    valid = (jpos >= left) & (jpos < right)
    scores = jnp.where(valid, scores, -1e30)
    m = jnp.max(scores, axis=1, keepdims=True)
    e = jnp.exp(scores - m)
    align = e / jnp.sum(e, axis=1, keepdims=True)
    d = jpos.astype(f32) - pt
    ex_p = jnp.exp(-(d * d) / (D * D / 2.0))
    at16 = (align * ex_p * valid.astype(f32)).astype(bf16)

    ct = jax.lax.dot_general(
        at16, g16, (((1,), (1,)), ((0,), (0,))),
        preferred_element_type=f32).astype(bf16)
    wcp.wait()
    out_ref[:] = jnp.dot(ct, wct_vmem[:].astype(bf16),
                         preferred_element_type=f32)


def kernel(encode_h, yt, encode_len, W_tan, w_pt, W_ct2ht):
    return pl.pallas_call(
        _nmt_kernel,
        in_specs=[
            pl.BlockSpec(memory_space=pl.ANY),
            pl.BlockSpec(memory_space=pltpu.MemorySpace.VMEM),
            pl.BlockSpec(memory_space=pltpu.MemorySpace.VMEM),
            pl.BlockSpec(memory_space=pltpu.MemorySpace.VMEM),
            pl.BlockSpec(memory_space=pltpu.MemorySpace.VMEM),
            pl.BlockSpec(memory_space=pl.ANY),
        ],
        out_shape=jax.ShapeDtypeStruct((B, H), f32),
        scratch_shapes=[
            pltpu.VMEM((B, WP, H), f32),
            pltpu.VMEM((B, 1), jnp.int32),
            pltpu.SMEM((B, 1), jnp.int32),
            pltpu.VMEM((H, H), f32),
            pltpu.SemaphoreType.DMA,
            pltpu.SemaphoreType.DMA,
            pltpu.SemaphoreType.DMA((B,)),
        ],
    )(encode_h, yt, encode_len.reshape(B, 1), W_tan, w_pt, W_ct2ht)
